# Initial kernel scaffold; baseline (speedup 1.0000x reference)
import jax
import jax.numpy as jnp
from jax.experimental import pallas as pl


def kernel(inputs, target):
    raise NotImplementedError("write your pallas kernel here")


# speedup vs baseline: 47.5094x; 47.5094x over previous
import functools

import jax
import jax.numpy as jnp
from jax import lax
from jax.experimental import pallas as pl
from jax.experimental.pallas import tpu as pltpu
from jax.experimental.pallas import tpu_sc as plsc

NCLS = 19
NPIX = 8 * 512 * 512
PLANE = 512 * 512
LANES = 16
NWORK = 32
NPW = NPIX // NWORK
CHUNK = 4096
NCHUNK = NPW // CHUNK
NBINS = 1536
STRIDE = NBINS + 1
HSIZE = LANES * STRIDE


def _hist_body(x_hbm, t_hbm, out_hbm, lbl_v, pbuf_v, hist_v, outbuf_v,
               sem0, sem1):
    wid = lax.axis_index("s") * 2 + lax.axis_index("c")
    pix_base = wid * NPW
    batch = wid // 4
    inb = (wid % 4) * NPW

    pltpu.sync_copy(t_hbm.at[pl.ds(pix_base, NPW)], lbl_v)

    zeros16 = jnp.zeros((LANES,), jnp.float32)
    ones16 = jnp.ones((LANES,), jnp.float32)
    lane_base = lax.iota(jnp.int32, LANES) * STRIDE

    def zero_body(j, carry):
        hist_v[pl.ds(j * LANES, LANES)] = zeros16
        return carry

    lax.fori_loop(0, (2 * HSIZE) // LANES, zero_body, 0)

    def class_body(c, carry):
        plane0 = (batch * NCLS + c) * PLANE + inb

        def compute_chunk(k, bufbase):
            def ibody(i, icarry):
                for u in range(4):
                    o = i * (4 * LANES) + u * LANES
                    p = pbuf_v[pl.ds(bufbase + o, LANES)]
                    lbl = lbl_v[pl.ds(k * CHUNK + o, LANES)]
                    fgf = jnp.where(lbl == c, ones16, zeros16)
                    e = jnp.abs(fgf - p)
                    bin_ = jnp.minimum((e * jnp.float32(NBINS)).astype(jnp.int32),
                                       NBINS - 1)
                    idx = lane_base + bin_
                    plsc.addupdate_scatter(hist_v, [idx], ones16)
                    plsc.addupdate_scatter(hist_v, [idx + HSIZE], fgf)
                return icarry

            lax.fori_loop(0, CHUNK // (4 * LANES), ibody, 0)

        buf0 = pbuf_v.at[pl.ds(0, CHUNK)]
        buf1 = pbuf_v.at[pl.ds(CHUNK, CHUNK)]
        pltpu.async_copy(x_hbm.at[pl.ds(plane0, CHUNK)], buf0, sem0)

        def pair_body(kk, pcarry):
            k0 = kk * 2
            pltpu.async_copy(
                x_hbm.at[pl.ds(plane0 + (k0 + 1) * CHUNK, CHUNK)],
                buf1, sem1)
            pltpu.make_async_copy(
                x_hbm.at[pl.ds(plane0 + k0 * CHUNK, CHUNK)],
                buf0, sem0).wait()
            compute_chunk(k0, 0)

            @pl.when(kk < NCHUNK // 2 - 1)
            def _():
                pltpu.async_copy(
                    x_hbm.at[pl.ds(plane0 + (k0 + 2) * CHUNK, CHUNK)],
                    buf0, sem0)

            pltpu.make_async_copy(
                x_hbm.at[pl.ds(plane0 + (k0 + 1) * CHUNK, CHUNK)],
                buf1, sem1).wait()
            compute_chunk(k0 + 1, CHUNK)
            return pcarry

        lax.fori_loop(0, NCHUNK // 2, pair_body, 0)

        def reduce_body(j, rcarry):
            o = j * LANES
            for h in range(2):
                hb = h * HSIZE
                acc = zeros16
                for l in range(LANES):
                    adr = hb + l * STRIDE + o
                    v = hist_v[pl.ds(adr, LANES)]
                    hist_v[pl.ds(adr, LANES)] = zeros16
                    acc = acc + v
                outbuf_v[pl.ds(h * NBINS + o, LANES)] = acc
            return rcarry

        lax.fori_loop(0, NBINS // LANES, reduce_body, 0)

        pltpu.sync_copy(
            outbuf_v,
            out_hbm.at[pl.ds((wid * NCLS + c) * 2 * NBINS, 2 * NBINS)])
        return carry

    lax.fori_loop(0, NCLS, class_body, 0)


_hist_call = functools.partial(
    pl.kernel,
    out_type=jax.ShapeDtypeStruct((NWORK * NCLS * 2 * NBINS,), jnp.float32),
    mesh=plsc.VectorSubcoreMesh(core_axis_name="c", subcore_axis_name="s"),
    compiler_params=pltpu.CompilerParams(needs_layout_passes=False),
    scratch_types=[
        pltpu.VMEM((NPW,), jnp.int32),
        pltpu.VMEM((2 * CHUNK,), jnp.float32),
        pltpu.VMEM((2 * HSIZE,), jnp.float32),
        pltpu.VMEM((2 * NBINS,), jnp.float32),
        pltpu.SemaphoreType.DMA,
        pltpu.SemaphoreType.DMA,
    ],
)(_hist_body)


def _scan_body(cnt_ref, pos_ref, out_ref):
    cnt = jnp.sum(cnt_ref[...], axis=0)
    pos = jnp.sum(pos_ref[...], axis=0)

    def rcum(x):
        y = x
        s = 1
        while s < NBINS:
            shifted = jnp.concatenate(
                [y[:, s:], jnp.zeros((NCLS, s), jnp.float32)], axis=1)
            y = y + shifted
            s *= 2
        return y

    n_incl = rcum(cnt)
    p_incl = rcum(pos)
    n_excl = n_incl - cnt
    p_excl = p_incl - pos
    g = p_incl[:, 0:1]

    def jac(n, p):
        return 1.0 - (g - p) / jnp.maximum(g + n - p, 1.0)

    emid = (lax.broadcasted_iota(jnp.int32, (NCLS, NBINS), 1).astype(
        jnp.float32) + 0.5) * (1.0 / NBINS)
    losses = jnp.sum(emid * (jac(n_incl, p_incl) - jac(n_excl, p_excl)),
                     axis=1, keepdims=True)
    present = (g > 0.0).astype(jnp.float32)
    total = jnp.sum(losses * present) / jnp.maximum(jnp.sum(present), 1.0)
    out_ref[...] = jnp.reshape(total, (1, 1))


_scan_call = pl.pallas_call(
    _scan_body,
    out_shape=jax.ShapeDtypeStruct((1, 1), jnp.float32),
)


def kernel(inputs, target):
    x = inputs.reshape(-1)
    t = target.reshape(-1)
    parts = _hist_call(x, t).reshape(NWORK, NCLS, 2, NBINS)
    out = _scan_call(parts[:, :, 0, :], parts[:, :, 1, :])
    return out[0, 0]


# speedup vs baseline: 132.1760x; 2.7821x over previous
import functools

import jax
import jax.numpy as jnp
from jax import lax
from jax.experimental import pallas as pl
from jax.experimental.pallas import tpu as pltpu
from jax.experimental.pallas import tpu_sc as plsc

NCLS = 19
NPIX = 8 * 512 * 512
PLANE = 512 * 512
LANES = 16
NWORK = 32
NPW = NPIX // NWORK
CHUNK = 4096
NCHUNK = NPW // CHUNK
NBINS = 1536
STRIDE = NBINS + 1
HSIZE = LANES * STRIDE


def _hist_body(x_hbm, t_hbm, out_hbm, lbl_v, pbuf_v, hist_v, outbuf_v,
               sem0, sem1):
    wid = lax.axis_index("s") * 2 + lax.axis_index("c")
    pix_base = wid * NPW
    batch = wid // 4
    inb = (wid % 4) * NPW

    pltpu.sync_copy(t_hbm.at[pl.ds(pix_base, NPW)], lbl_v)

    zeros16 = jnp.zeros((LANES,), jnp.float32)
    ones16 = jnp.ones((LANES,), jnp.float32)
    lane_base = lax.iota(jnp.int32, LANES) * STRIDE

    def _zero_body(j, cval):
        hist_v[pl.ds(j * LANES, LANES)] = zeros16
        return cval

    zdep = plsc.parallel_loop(0, (2 * HSIZE) // LANES, unroll=8,
                              carry=jnp.int32(0))(_zero_body)

    def class_body(c, carry):
        plane0 = (batch * NCLS + c) * PLANE + inb

        def compute_chunk(k, bufbase):
            def _main(i, cval):
                o = i * LANES
                p = pbuf_v[pl.ds(bufbase + o, LANES)]
                lbl = lbl_v[pl.ds(k * CHUNK + o, LANES)]
                fgf = jnp.where(lbl == c, ones16, zeros16)
                e = jnp.abs(fgf - p)
                binf = jnp.minimum(e * jnp.float32(NBINS),
                                   jnp.float32(NBINS - 1))
                bin_ = binf.astype(jnp.int32)
                idx = lane_base + bin_
                plsc.addupdate_scatter(hist_v, [idx], ones16)
                plsc.addupdate_scatter(hist_v, [idx + HSIZE], fgf)
                return cval

            return plsc.parallel_loop(0, CHUNK // LANES, unroll=8,
                                      carry=jnp.int32(0))(_main)

        buf0 = pbuf_v.at[pl.ds(0, CHUNK)]
        buf1 = pbuf_v.at[pl.ds(CHUNK, CHUNK)]
        pltpu.async_copy(x_hbm.at[pl.ds(plane0, CHUNK)], buf0, sem0)

        def pair_body(kk, pcarry):
            k0 = kk * 2
            pltpu.async_copy(
                x_hbm.at[pl.ds(plane0 + (k0 + 1) * CHUNK, CHUNK)],
                buf1, sem1)
            pltpu.make_async_copy(
                x_hbm.at[pl.ds(plane0 + k0 * CHUNK, CHUNK)],
                buf0, sem0).wait()
            pcarry = pcarry + compute_chunk(k0, 0)

            @pl.when(kk < NCHUNK // 2 - 1)
            def _():
                pltpu.async_copy(
                    x_hbm.at[pl.ds(plane0 + (k0 + 2) * CHUNK, CHUNK)],
                    buf0, sem0)

            pltpu.make_async_copy(
                x_hbm.at[pl.ds(plane0 + (k0 + 1) * CHUNK, CHUNK)],
                buf1, sem1).wait()
            pcarry = pcarry + compute_chunk(k0 + 1, CHUNK)
            return pcarry

        dep = lax.fori_loop(0, NCHUNK // 2, pair_body, zdep + carry)

        def _reduce_body(j, cval):
            o = j * LANES
            for h in range(2):
                hb = h * HSIZE
                acc = zeros16
                for l in range(LANES):
                    adr = hb + l * STRIDE + o
                    v = hist_v[pl.ds(adr, LANES)]
                    hist_v[pl.ds(adr, LANES)] = zeros16
                    acc = acc + v
                outbuf_v[pl.ds(h * NBINS + o, LANES)] = acc
            return cval

        rdep = plsc.parallel_loop(0, NBINS // LANES, unroll=2,
                                  carry=dep)(_reduce_body)
        base = (wid * NCLS + c) * 2 * NBINS + jnp.minimum(rdep, 0)
        pltpu.sync_copy(
            outbuf_v,
            out_hbm.at[pl.ds(base, 2 * NBINS)])
        return rdep

    lax.fori_loop(0, NCLS, class_body, jnp.int32(0))


_hist_call = functools.partial(
    pl.kernel,
    out_type=jax.ShapeDtypeStruct((NWORK * NCLS * 2 * NBINS,), jnp.float32),
    mesh=plsc.VectorSubcoreMesh(core_axis_name="c", subcore_axis_name="s"),
    compiler_params=pltpu.CompilerParams(needs_layout_passes=False),
    scratch_types=[
        pltpu.VMEM((NPW,), jnp.int32),
        pltpu.VMEM((2 * CHUNK,), jnp.float32),
        pltpu.VMEM((2 * HSIZE,), jnp.float32),
        pltpu.VMEM((2 * NBINS,), jnp.float32),
        pltpu.SemaphoreType.DMA,
        pltpu.SemaphoreType.DMA,
    ],
)(_hist_body)


def _scan_body(cnt_ref, pos_ref, out_ref):
    cnt = jnp.sum(cnt_ref[...], axis=0)
    pos = jnp.sum(pos_ref[...], axis=0)

    def rcum(x):
        y = x
        s = 1
        while s < NBINS:
            shifted = jnp.concatenate(
                [y[:, s:], jnp.zeros((NCLS, s), jnp.float32)], axis=1)
            y = y + shifted
            s *= 2
        return y

    n_incl = rcum(cnt)
    p_incl = rcum(pos)
    n_excl = n_incl - cnt
    p_excl = p_incl - pos
    g = p_incl[:, 0:1]

    def jac(n, p):
        return 1.0 - (g - p) / jnp.maximum(g + n - p, 1.0)

    emid = (lax.broadcasted_iota(jnp.int32, (NCLS, NBINS), 1).astype(
        jnp.float32) + 0.5) * (1.0 / NBINS)
    losses = jnp.sum(emid * (jac(n_incl, p_incl) - jac(n_excl, p_excl)),
                     axis=1, keepdims=True)
    present = (g > 0.0).astype(jnp.float32)
    total = jnp.sum(losses * present) / jnp.maximum(jnp.sum(present), 1.0)
    out_ref[...] = jnp.reshape(total, (1, 1))


_scan_call = pl.pallas_call(
    _scan_body,
    out_shape=jax.ShapeDtypeStruct((1, 1), jnp.float32),
)


def kernel(inputs, target):
    x = inputs.reshape(-1)
    t = target.reshape(-1)
    parts = _hist_call(x, t).reshape(NWORK, NCLS, 2, NBINS)
    out = _scan_call(parts[:, :, 0, :], parts[:, :, 1, :])
    return out[0, 0]


# speedup vs baseline: 183.3707x; 1.3873x over previous
import functools

import jax
import jax.numpy as jnp
from jax import lax
from jax.experimental import pallas as pl
from jax.experimental.pallas import tpu as pltpu
from jax.experimental.pallas import tpu_sc as plsc

NCLS = 19
NPIX = 8 * 512 * 512
PLANE = 512 * 512
LANES = 16
NWORK = 32
NPW = NPIX // NWORK
CHUNK = 8192
NCHUNK = NPW // CHUNK
NBINS = 2048
BSHIFT = 23 - 11
STRIDE = NBINS + 1
HSIZE = LANES * STRIDE


def _hist_body(x_hbm, t_hbm, out_hbm, lbl_v, pbuf_v, hist_v, outbuf_v,
               sem0, sem1):
    wid = lax.axis_index("s") * 2 + lax.axis_index("c")
    pix_base = wid * NPW
    batch = wid // 4
    inb = (wid % 4) * NPW

    pltpu.sync_copy(t_hbm.at[pl.ds(pix_base, NPW)], lbl_v)

    zeros16 = jnp.zeros((LANES,), jnp.float32)
    ones16 = jnp.ones((LANES,), jnp.float32)
    izeros = jnp.zeros((LANES,), jnp.int32)
    ipos = jnp.full((LANES,), 8193, jnp.int32)
    ione = jnp.ones((LANES,), jnp.int32)
    lane_base = lax.iota(jnp.int32, LANES) * STRIDE

    def _zero_body(j, cval):
        hist_v[pl.ds(j * LANES, LANES)] = izeros
        return cval

    zdep = plsc.parallel_loop(0, HSIZE // LANES, unroll=8,
                              carry=jnp.int32(0))(_zero_body)

    def class_body(c, carry):
        plane0 = (batch * NCLS + c) * PLANE + inb

        def compute_chunk(k, bufbase):
            def _main(i, cval):
                o = i * LANES
                p = pbuf_v[pl.ds(bufbase + o, LANES)]
                lbl = lbl_v[pl.ds(k * CHUNK + o, LANES)]
                m = lbl == c
                fgf = jnp.where(m, ones16, zeros16)
                q = jnp.abs(fgf - p) + 1.0
                bits = plsc.bitcast(q, jnp.int32)
                bin_ = (bits >> BSHIFT) & (NBINS - 1)
                idx = lane_base + bin_
                val = jnp.where(m, ipos, ione)
                plsc.addupdate_scatter(hist_v, [idx], val)
                return cval

            return plsc.parallel_loop(0, CHUNK // LANES, unroll=8,
                                      carry=jnp.int32(0))(_main)

        buf0 = pbuf_v.at[pl.ds(0, CHUNK)]
        buf1 = pbuf_v.at[pl.ds(CHUNK, CHUNK)]
        pltpu.async_copy(x_hbm.at[pl.ds(plane0, CHUNK)], buf0, sem0)

        def pair_body(kk, pcarry):
            k0 = kk * 2
            pltpu.async_copy(
                x_hbm.at[pl.ds(plane0 + (k0 + 1) * CHUNK, CHUNK)],
                buf1, sem1)
            pltpu.make_async_copy(
                x_hbm.at[pl.ds(plane0 + k0 * CHUNK, CHUNK)],
                buf0, sem0).wait()
            pcarry = pcarry + compute_chunk(k0, 0)

            @pl.when(kk < NCHUNK // 2 - 1)
            def _():
                pltpu.async_copy(
                    x_hbm.at[pl.ds(plane0 + (k0 + 2) * CHUNK, CHUNK)],
                    buf0, sem0)

            pltpu.make_async_copy(
                x_hbm.at[pl.ds(plane0 + (k0 + 1) * CHUNK, CHUNK)],
                buf1, sem1).wait()
            pcarry = pcarry + compute_chunk(k0 + 1, CHUNK)
            return pcarry

        dep = lax.fori_loop(0, NCHUNK // 2, pair_body, zdep + carry)

        def _reduce_body(j, cval):
            o = j * LANES
            acc_c = izeros
            acc_p = izeros
            for l in range(LANES):
                adr = l * STRIDE + o
                v = hist_v[pl.ds(adr, LANES)]
                hist_v[pl.ds(adr, LANES)] = izeros
                acc_c = acc_c + (v & 8191)
                acc_p = acc_p + (v >> 13)
            outbuf_v[pl.ds(o, LANES)] = acc_c
            outbuf_v[pl.ds(NBINS + o, LANES)] = acc_p
            return cval

        rdep = plsc.parallel_loop(0, NBINS // LANES, unroll=2,
                                  carry=dep)(_reduce_body)
        base = (wid * NCLS + c) * 2 * NBINS + jnp.minimum(rdep, 0)
        pltpu.sync_copy(
            outbuf_v,
            out_hbm.at[pl.ds(base, 2 * NBINS)])
        return rdep

    lax.fori_loop(0, NCLS, class_body, jnp.int32(0))


_hist_call = functools.partial(
    pl.kernel,
    out_type=jax.ShapeDtypeStruct((NWORK * NCLS * 2 * NBINS,), jnp.int32),
    mesh=plsc.VectorSubcoreMesh(core_axis_name="c", subcore_axis_name="s"),
    compiler_params=pltpu.CompilerParams(needs_layout_passes=False),
    scratch_types=[
        pltpu.VMEM((NPW,), jnp.int32),
        pltpu.VMEM((2 * CHUNK,), jnp.float32),
        pltpu.VMEM((HSIZE,), jnp.int32),
        pltpu.VMEM((2 * NBINS,), jnp.int32),
        pltpu.SemaphoreType.DMA,
        pltpu.SemaphoreType.DMA,
    ],
)(_hist_body)


def _scan_body(cnt_ref, pos_ref, out_ref):
    cnt = jnp.sum(cnt_ref[...], axis=0).astype(jnp.float32)
    pos = jnp.sum(pos_ref[...], axis=0).astype(jnp.float32)

    def rcum(x):
        y = x
        s = 1
        while s < NBINS:
            shifted = jnp.concatenate(
                [y[:, s:], jnp.zeros((NCLS, s), jnp.float32)], axis=1)
            y = y + shifted
            s *= 2
        return y

    n_incl = rcum(cnt)
    p_incl = rcum(pos)
    n_excl = n_incl - cnt
    p_excl = p_incl - pos
    g = p_incl[:, 0:1]

    def jac(n, p):
        return 1.0 - (g - p) / jnp.maximum(g + n - p, 1.0)

    emid = (lax.broadcasted_iota(jnp.int32, (NCLS, NBINS), 1).astype(
        jnp.float32) + 0.5) * (1.0 / NBINS)
    losses = jnp.sum(emid * (jac(n_incl, p_incl) - jac(n_excl, p_excl)),
                     axis=1, keepdims=True)
    present = (g > 0.0).astype(jnp.float32)
    total = jnp.sum(losses * present) / jnp.maximum(jnp.sum(present), 1.0)
    out_ref[...] = jnp.reshape(total, (1, 1))


_scan_call = pl.pallas_call(
    _scan_body,
    out_shape=jax.ShapeDtypeStruct((1, 1), jnp.float32),
)


def kernel(inputs, target):
    x = inputs.reshape(-1)
    t = target.reshape(-1)
    parts = _hist_call(x, t).reshape(NWORK, NCLS, 2, NBINS)
    out = _scan_call(parts[:, :, 0, :], parts[:, :, 1, :])
    return out[0, 0]


# speedup vs baseline: 195.5513x; 1.0664x over previous
import functools

import jax
import jax.numpy as jnp
from jax import lax
from jax.experimental import pallas as pl
from jax.experimental.pallas import tpu as pltpu
from jax.experimental.pallas import tpu_sc as plsc

NCLS = 19
NPIX = 8 * 512 * 512
PLANE = 512 * 512
LANES = 16
NWORK = 32
NPW = NPIX // NWORK
CHUNK = 8192
NCHUNK = NPW // CHUNK
NBINS = 2048
BSHIFT = 23 - 11
STRIDE = NBINS + 1
HSIZE = LANES * STRIDE


def _hist_body(x_hbm, t_hbm, out_hbm, lbl_v, pbuf_v, hist_v, outbuf_v,
               sem0, sem1, sem2):
    wid = lax.axis_index("s") * 2 + lax.axis_index("c")
    pix_base = wid * NPW
    batch = wid // 4
    inb = (wid % 4) * NPW

    pltpu.sync_copy(t_hbm.at[pl.ds(pl.multiple_of(pix_base, NPW), NPW)],
                    lbl_v)

    zeros16 = jnp.zeros((LANES,), jnp.float32)
    ones16 = jnp.ones((LANES,), jnp.float32)
    izeros = jnp.zeros((LANES,), jnp.int32)
    ipos = jnp.full((LANES,), 8193, jnp.int32)
    ione = jnp.ones((LANES,), jnp.int32)
    lane_base = lax.iota(jnp.int32, LANES) * STRIDE

    def _zero_body(j, cval):
        hist_v[pl.ds(j * LANES, LANES)] = izeros
        return cval

    zdep = plsc.parallel_loop(0, HSIZE // LANES, unroll=8,
                              carry=jnp.int32(0))(_zero_body)

    TOTAL = NCLS * NCHUNK

    def addr(s):
        cc = s >> 3
        kk = s & (NCHUNK - 1)
        return pl.multiple_of(
            (batch * NCLS + cc) * PLANE + inb + kk * CHUNK, CHUNK)

    buf0 = pbuf_v.at[pl.ds(0, CHUNK)]
    buf1 = pbuf_v.at[pl.ds(CHUNK, CHUNK)]

    def start(s, buf, sem):
        pltpu.async_copy(x_hbm.at[pl.ds(addr(s), CHUNK)], buf, sem)

    def wait(s, buf, sem):
        pltpu.make_async_copy(x_hbm.at[pl.ds(addr(s), CHUNK)], buf,
                              sem).wait()

    def compute_chunk(s, bufbase):
        loff = (s & (NCHUNK - 1)) * CHUNK
        cval_cls = s >> 3

        def _main(i, cval):
            o = i * LANES
            p = pbuf_v[pl.ds(bufbase + o, LANES)]
            lbl = lbl_v[pl.ds(loff + o, LANES)]
            m = lbl == cval_cls
            fgf = jnp.where(m, ones16, zeros16)
            q = jnp.abs(fgf - p) + 1.0
            bits = plsc.bitcast(q, jnp.int32)
            bin_ = (bits >> BSHIFT) & (NBINS - 1)
            idx = lane_base + bin_
            val = jnp.where(m, ipos, ione)
            plsc.addupdate_scatter(hist_v, [idx], val)
            return cval

        return plsc.parallel_loop(0, CHUNK // LANES, unroll=8,
                                  carry=jnp.int32(0))(_main)

    def _reduce_body(j, cval):
        o = j * LANES
        acc_c = izeros
        acc_p = izeros
        for l in range(LANES):
            adr = l * STRIDE + o
            v = hist_v[pl.ds(adr, LANES)]
            hist_v[pl.ds(adr, LANES)] = izeros
            acc_c = acc_c + (v & 8191)
            acc_p = acc_p + (v >> 13)
        outbuf_v[pl.ds(o, LANES)] = acc_c
        outbuf_v[pl.ds(NBINS + o, LANES)] = acc_p
        return cval

    def boundary(s, dep):
        cc = s >> 3

        @pl.when((s & (NCHUNK - 1)) == NCHUNK - 1)
        def _():
            @pl.when(cc > 0)
            def _():
                pltpu.make_async_copy(
                    outbuf_v, out_hbm.at[pl.ds(0, 2 * NBINS)], sem2).wait()

            rdep = plsc.parallel_loop(0, NBINS // LANES, unroll=2,
                                      carry=dep)(_reduce_body)
            base = pl.multiple_of(
                (wid * NCLS + cc) * 2 * NBINS + jnp.minimum(rdep, 0), 16)
            pltpu.async_copy(outbuf_v, out_hbm.at[pl.ds(base, 2 * NBINS)],
                             sem2)

    start(0, buf0, sem0)
    start(1, buf1, sem1)

    def step2_body(s2, carry):
        s0 = s2 * 2
        s1 = s0 + 1
        wait(s0, buf0, sem0)
        carry = carry + compute_chunk(s0, 0)

        @pl.when(s0 + 2 < TOTAL)
        def _():
            start(s0 + 2, buf0, sem0)

        wait(s1, buf1, sem1)
        carry = carry + compute_chunk(s1, CHUNK)

        @pl.when(s1 + 2 < TOTAL)
        def _():
            start(s1 + 2, buf1, sem1)

        boundary(s1, carry)
        return carry

    lax.fori_loop(0, TOTAL // 2, step2_body, zdep)

    pltpu.make_async_copy(outbuf_v, out_hbm.at[pl.ds(0, 2 * NBINS)],
                          sem2).wait()


_hist_call = functools.partial(
    pl.kernel,
    out_type=jax.ShapeDtypeStruct((NWORK * NCLS * 2 * NBINS,), jnp.int32),
    mesh=plsc.VectorSubcoreMesh(core_axis_name="c", subcore_axis_name="s"),
    compiler_params=pltpu.CompilerParams(needs_layout_passes=False),
    scratch_types=[
        pltpu.VMEM((NPW,), jnp.int32),
        pltpu.VMEM((2 * CHUNK,), jnp.float32),
        pltpu.VMEM((HSIZE,), jnp.int32),
        pltpu.VMEM((2 * NBINS,), jnp.int32),
        pltpu.SemaphoreType.DMA,
        pltpu.SemaphoreType.DMA,
        pltpu.SemaphoreType.DMA,
    ],
)(_hist_body)


def _scan_body(cnt_ref, pos_ref, out_ref):
    cnt = jnp.sum(cnt_ref[...], axis=0).astype(jnp.float32)
    pos = jnp.sum(pos_ref[...], axis=0).astype(jnp.float32)

    def rcum(x):
        y = x
        s = 1
        while s < NBINS:
            shifted = jnp.concatenate(
                [y[:, s:], jnp.zeros((NCLS, s), jnp.float32)], axis=1)
            y = y + shifted
            s *= 2
        return y

    n_incl = rcum(cnt)
    p_incl = rcum(pos)
    n_excl = n_incl - cnt
    p_excl = p_incl - pos
    g = p_incl[:, 0:1]

    def jac(n, p):
        return 1.0 - (g - p) / jnp.maximum(g + n - p, 1.0)

    emid = (lax.broadcasted_iota(jnp.int32, (NCLS, NBINS), 1).astype(
        jnp.float32) + 0.5) * (1.0 / NBINS)
    losses = jnp.sum(emid * (jac(n_incl, p_incl) - jac(n_excl, p_excl)),
                     axis=1, keepdims=True)
    present = (g > 0.0).astype(jnp.float32)
    total = jnp.sum(losses * present) / jnp.maximum(jnp.sum(present), 1.0)
    out_ref[...] = jnp.reshape(total, (1, 1))


_scan_call = pl.pallas_call(
    _scan_body,
    out_shape=jax.ShapeDtypeStruct((1, 1), jnp.float32),
)


def kernel(inputs, target):
    x = inputs.reshape(-1)
    t = target.reshape(-1)
    parts = _hist_call(x, t).reshape(NWORK, NCLS, 2, NBINS)
    out = _scan_call(parts[:, :, 0, :], parts[:, :, 1, :])
    return out[0, 0]


# speedup vs baseline: 210.2479x; 1.0752x over previous
import functools

import jax
import jax.numpy as jnp
from jax import lax
from jax.experimental import pallas as pl
from jax.experimental.pallas import tpu as pltpu
from jax.experimental.pallas import tpu_sc as plsc

NCLS = 19
NPIX = 8 * 512 * 512
PLANE = 512 * 512
LANES = 16
NWORK = 32
NPW = NPIX // NWORK
CHUNK = 8192
NCHUNK = NPW // CHUNK
NBINS = 2048
BSHIFT = 23 - 11
STRIDE = NBINS + 1
HSIZE = LANES * STRIDE


def _hist_body(x_hbm, t_hbm, out_hbm, lbl_v, pbuf_v, hist_v, outbuf_v,
               sem0, sem1, sem2):
    wid = lax.axis_index("s") * 2 + lax.axis_index("c")
    pix_base = wid * NPW
    batch = wid // 4
    inb = (wid % 4) * NPW

    pltpu.sync_copy(t_hbm.at[wid], lbl_v)

    zeros16 = jnp.zeros((LANES,), jnp.float32)
    ones16 = jnp.ones((LANES,), jnp.float32)
    izeros = jnp.zeros((LANES,), jnp.int32)
    ipos = jnp.full((LANES,), 8193, jnp.int32)
    ione = jnp.ones((LANES,), jnp.int32)

    def _zero_body(j, cval):
        hist_v[pl.ds(j * LANES, LANES)] = izeros
        return cval

    zdep = plsc.parallel_loop(0, NBINS // LANES, unroll=8,
                              carry=jnp.int32(0))(_zero_body)

    TOTAL = NCLS * NCHUNK

    def addr(s):
        cc = s >> 3
        kk = s & (NCHUNK - 1)
        return pl.multiple_of(
            (batch * NCLS + cc) * PLANE + inb + kk * CHUNK, CHUNK)

    buf0 = pbuf_v.at[pl.ds(0, CHUNK)]
    buf1 = pbuf_v.at[pl.ds(CHUNK, CHUNK)]

    def start(s, buf, sem):
        pltpu.async_copy(x_hbm.at[pl.ds(addr(s), CHUNK)], buf, sem)

    def wait(s, buf, sem):
        pltpu.make_async_copy(x_hbm.at[pl.ds(addr(s), CHUNK)], buf,
                              sem).wait()

    def compute_chunk(s, bufbase):
        loff = (s & (NCHUNK - 1)) * CHUNK
        cval_cls = s >> 3

        def _main(i, cval):
            o = i * LANES
            p = pbuf_v[pl.ds(bufbase + o, LANES)]
            lbl = lbl_v[pl.ds(loff + o, LANES)]
            m = lbl == cval_cls
            q = jnp.where(m, 2.0 - p, 1.0 + p)
            bits = plsc.bitcast(q, jnp.int32)
            bin_ = (bits >> BSHIFT) & (NBINS - 1)
            val = jnp.where(m, ipos, ione)
            plsc.addupdate_scatter(hist_v, [bin_], val)
            return cval

        return plsc.parallel_loop(0, CHUNK // LANES, unroll=8,
                                  carry=jnp.int32(0))(_main)

    def _reduce_body(j, cval):
        o = j * LANES
        v = hist_v[pl.ds(o, LANES)]
        hist_v[pl.ds(o, LANES)] = izeros
        outbuf_v[pl.ds(o, LANES)] = v & 8191
        outbuf_v[pl.ds(NBINS + o, LANES)] = v >> 13
        return cval

    def boundary(s, dep):
        cc = s >> 3

        @pl.when((s & (NCHUNK - 1)) == NCHUNK - 1)
        def _():
            @pl.when(cc > 0)
            def _():
                pltpu.make_async_copy(outbuf_v, out_hbm.at[0], sem2).wait()

            rdep = plsc.parallel_loop(0, NBINS // LANES, unroll=2,
                                      carry=dep)(_reduce_body)
            base = wid * NCLS + cc + jnp.minimum(rdep, 0)
            pltpu.async_copy(outbuf_v, out_hbm.at[base], sem2)

    start(0, buf0, sem0)
    start(1, buf1, sem1)

    def step2_body(s2, carry):
        s0 = s2 * 2
        s1 = s0 + 1
        wait(s0, buf0, sem0)
        carry = carry + compute_chunk(s0, 0)

        @pl.when(s0 + 2 < TOTAL)
        def _():
            start(s0 + 2, buf0, sem0)

        wait(s1, buf1, sem1)
        carry = carry + compute_chunk(s1, CHUNK)

        @pl.when(s1 + 2 < TOTAL)
        def _():
            start(s1 + 2, buf1, sem1)

        boundary(s1, carry)
        return carry

    lax.fori_loop(0, TOTAL // 2, step2_body, zdep)

    pltpu.make_async_copy(outbuf_v, out_hbm.at[0], sem2).wait()


_hist_call = functools.partial(
    pl.kernel,
    out_type=jax.ShapeDtypeStruct((NWORK * NCLS, 2 * NBINS), jnp.int32),
    mesh=plsc.VectorSubcoreMesh(core_axis_name="c", subcore_axis_name="s"),
    compiler_params=pltpu.CompilerParams(needs_layout_passes=False),
    scratch_types=[
        pltpu.VMEM((NPW,), jnp.int32),
        pltpu.VMEM((2 * CHUNK,), jnp.float32),
        pltpu.VMEM((NBINS,), jnp.int32),
        pltpu.VMEM((2 * NBINS,), jnp.int32),
        pltpu.SemaphoreType.DMA,
        pltpu.SemaphoreType.DMA,
        pltpu.SemaphoreType.DMA,
    ],
)(_hist_body)


def _scan_body(cnt_ref, pos_ref, out_ref):
    cnt = jnp.sum(cnt_ref[...], axis=0).astype(jnp.float32)
    pos = jnp.sum(pos_ref[...], axis=0).astype(jnp.float32)

    def rcum(x):
        y = x
        s = 1
        while s < NBINS:
            shifted = jnp.concatenate(
                [y[:, s:], jnp.zeros((NCLS, s), jnp.float32)], axis=1)
            y = y + shifted
            s *= 2
        return y

    n_incl = rcum(cnt)
    p_incl = rcum(pos)
    n_excl = n_incl - cnt
    p_excl = p_incl - pos
    g = p_incl[:, 0:1]

    def jac(n, p):
        return 1.0 - (g - p) / jnp.maximum(g + n - p, 1.0)

    emid = (lax.broadcasted_iota(jnp.int32, (NCLS, NBINS), 1).astype(
        jnp.float32) + 0.5) * (1.0 / NBINS)
    losses = jnp.sum(emid * (jac(n_incl, p_incl) - jac(n_excl, p_excl)),
                     axis=1, keepdims=True)
    present = (g > 0.0).astype(jnp.float32)
    total = jnp.sum(losses * present) / jnp.maximum(jnp.sum(present), 1.0)
    out_ref[...] = jnp.reshape(total, (1, 1))


_scan_call = pl.pallas_call(
    _scan_body,
    out_shape=jax.ShapeDtypeStruct((1, 1), jnp.float32),
)


def kernel(inputs, target):
    x = inputs.reshape(-1)
    t = target.reshape(NWORK, NPW)
    parts = _hist_call(x, t).reshape(NWORK, NCLS, 2, NBINS)
    out = _scan_call(parts[:, :, 0, :], parts[:, :, 1, :])
    return out[0, 0]


# speedup vs baseline: 212.4779x; 1.0106x over previous
import functools

import jax
import jax.numpy as jnp
from jax import lax
from jax.experimental import pallas as pl
from jax.experimental.pallas import tpu as pltpu
from jax.experimental.pallas import tpu_sc as plsc

NCLS = 19
NPIX = 8 * 512 * 512
PLANE = 512 * 512
LANES = 16
NWORK = 32
NPW = NPIX // NWORK
CHUNK = 16384
NCHUNK = NPW // CHUNK
NBINS = 2048
BSHIFT = 23 - 11
STRIDE = NBINS + 1
HSIZE = LANES * STRIDE


def _hist_body(x_hbm, t_hbm, out_hbm, lbl_v, pbuf_v, hist_v, outbuf_v,
               sem0, sem1, sem2):
    wid = lax.axis_index("s") * 2 + lax.axis_index("c")
    pix_base = wid * NPW
    batch = wid // 4
    inb = (wid % 4) * NPW

    pltpu.sync_copy(t_hbm.at[wid], lbl_v)

    zeros16 = jnp.zeros((LANES,), jnp.float32)
    ones16 = jnp.ones((LANES,), jnp.float32)
    izeros = jnp.zeros((LANES,), jnp.int32)
    ipos = jnp.full((LANES,), 8193, jnp.int32)
    ione = jnp.ones((LANES,), jnp.int32)

    def _zero_body(j, cval):
        hist_v[pl.ds(j * LANES, LANES)] = izeros
        return cval

    zdep = plsc.parallel_loop(0, NBINS // LANES, unroll=8,
                              carry=jnp.int32(0))(_zero_body)

    TOTAL = NCLS * NCHUNK

    def addr(s):
        cc = s >> 2
        kk = s & (NCHUNK - 1)
        return pl.multiple_of(
            (batch * NCLS + cc) * PLANE + inb + kk * CHUNK, CHUNK)

    buf0 = pbuf_v.at[pl.ds(0, CHUNK)]
    buf1 = pbuf_v.at[pl.ds(CHUNK, CHUNK)]

    def start(s, buf, sem):
        pltpu.async_copy(x_hbm.at[pl.ds(addr(s), CHUNK)], buf, sem)

    def wait(s, buf, sem):
        pltpu.make_async_copy(x_hbm.at[pl.ds(addr(s), CHUNK)], buf,
                              sem).wait()

    def compute_chunk(s, bufbase):
        loff = (s & (NCHUNK - 1)) * CHUNK
        cval_cls = s >> 2

        def _main(i, cval):
            o = i * LANES
            p = pbuf_v[pl.ds(bufbase + o, LANES)]
            lbl = lbl_v[pl.ds(loff + o, LANES)]
            m = lbl == cval_cls
            q = jnp.where(m, 2.0 - p, 1.0 + p)
            bits = plsc.bitcast(q, jnp.int32)
            bin_ = (bits >> BSHIFT) - (127 << (23 - BSHIFT))
            val = jnp.where(m, ipos, ione)
            plsc.addupdate_scatter(hist_v, [bin_], val)
            return cval

        return plsc.parallel_loop(0, CHUNK // LANES, unroll=8,
                                  carry=jnp.int32(0))(_main)

    def _reduce_body(j, cval):
        o = j * LANES
        v = hist_v[pl.ds(o, LANES)]
        hist_v[pl.ds(o, LANES)] = izeros
        outbuf_v[pl.ds(o, LANES)] = v & 8191
        outbuf_v[pl.ds(NBINS + o, LANES)] = v >> 13
        return cval

    def boundary(s, dep):
        cc = s >> 2

        @pl.when((s & (NCHUNK - 1)) == NCHUNK - 1)
        def _():
            @pl.when(cc > 0)
            def _():
                pltpu.make_async_copy(outbuf_v, out_hbm.at[0], sem2).wait()

            rdep = plsc.parallel_loop(0, NBINS // LANES, unroll=2,
                                      carry=dep)(_reduce_body)
            base = wid * NCLS + cc + jnp.minimum(rdep, 0)
            pltpu.async_copy(outbuf_v, out_hbm.at[base], sem2)

    start(0, buf0, sem0)
    start(1, buf1, sem1)

    def step2_body(s2, carry):
        s0 = s2 * 2
        s1 = s0 + 1
        wait(s0, buf0, sem0)
        carry = carry + compute_chunk(s0, 0)

        @pl.when(s0 + 2 < TOTAL)
        def _():
            start(s0 + 2, buf0, sem0)

        wait(s1, buf1, sem1)
        carry = carry + compute_chunk(s1, CHUNK)

        @pl.when(s1 + 2 < TOTAL)
        def _():
            start(s1 + 2, buf1, sem1)

        boundary(s1, carry)
        return carry

    lax.fori_loop(0, TOTAL // 2, step2_body, zdep)

    pltpu.make_async_copy(outbuf_v, out_hbm.at[0], sem2).wait()


_hist_call = functools.partial(
    pl.kernel,
    out_type=jax.ShapeDtypeStruct((NWORK * NCLS, 2 * NBINS), jnp.int32),
    mesh=plsc.VectorSubcoreMesh(core_axis_name="c", subcore_axis_name="s"),
    compiler_params=pltpu.CompilerParams(needs_layout_passes=False),
    scratch_types=[
        pltpu.VMEM((NPW,), jnp.int32),
        pltpu.VMEM((2 * CHUNK,), jnp.float32),
        pltpu.VMEM((NBINS + 16,), jnp.int32),
        pltpu.VMEM((2 * NBINS,), jnp.int32),
        pltpu.SemaphoreType.DMA,
        pltpu.SemaphoreType.DMA,
        pltpu.SemaphoreType.DMA,
    ],
)(_hist_body)


def _scan_body(cnt_ref, pos_ref, out_ref):
    cnt = jnp.sum(cnt_ref[...], axis=0).astype(jnp.float32)
    pos = jnp.sum(pos_ref[...], axis=0).astype(jnp.float32)

    def rcum(x):
        y = x
        s = 1
        while s < NBINS:
            shifted = jnp.concatenate(
                [y[:, s:], jnp.zeros((NCLS, s), jnp.float32)], axis=1)
            y = y + shifted
            s *= 2
        return y

    n_incl = rcum(cnt)
    p_incl = rcum(pos)
    n_excl = n_incl - cnt
    p_excl = p_incl - pos
    g = p_incl[:, 0:1]

    def jac(n, p):
        return 1.0 - (g - p) / jnp.maximum(g + n - p, 1.0)

    emid = (lax.broadcasted_iota(jnp.int32, (NCLS, NBINS), 1).astype(
        jnp.float32) + 0.5) * (1.0 / NBINS)
    losses = jnp.sum(emid * (jac(n_incl, p_incl) - jac(n_excl, p_excl)),
                     axis=1, keepdims=True)
    present = (g > 0.0).astype(jnp.float32)
    total = jnp.sum(losses * present) / jnp.maximum(jnp.sum(present), 1.0)
    out_ref[...] = jnp.reshape(total, (1, 1))


_scan_call = pl.pallas_call(
    _scan_body,
    out_shape=jax.ShapeDtypeStruct((1, 1), jnp.float32),
)


def kernel(inputs, target):
    x = inputs.reshape(-1)
    t = target.reshape(NWORK, NPW)
    parts = _hist_call(x, t).reshape(NWORK, NCLS, 2, NBINS)
    out = _scan_call(parts[:, :, 0, :], parts[:, :, 1, :])
    return out[0, 0]
